# baseline (device time: 55997 ns/iter reference)
import jax
import jax.numpy as jnp
from jax import lax
from jax.experimental import pallas as pl
from jax.experimental.pallas import tpu as pltpu

N_DEV = 4
B = 16
H = 16
D = 64
BS = 16
NB = 128
P_LOCAL = 128
NKEYS = P_LOCAL * BS
R = B * H
HD = H * D
NEG = -1e30


def _body(qbd_ref, k_ref, v_ref, cnt_ref, out_ref,
          mine_ref, comm_ref, send_sems, recv_sems):
    my = lax.axis_index("i")

    bsem = pltpu.get_barrier_semaphore()
    for k in (1, 2, 3):
        pl.semaphore_signal(
            bsem, inc=1,
            device_id=((my + k) % N_DEV,),
            device_id_type=pl.DeviceIdType.MESH,
        )
    pl.semaphore_wait(bsem, N_DEV - 1)

    e_row = lax.div(lax.broadcasted_iota(jnp.int32, (R, B), 0), H)
    e_col = lax.broadcasted_iota(jnp.int32, (R, B), 1)
    expand = (e_row == e_col).astype(jnp.bfloat16)
    cnt = lax.dot_general(
        expand, cnt_ref[:, :], (((1,), (0,)), ((), ())),
        preferred_element_type=jnp.float32,
    )

    s = lax.dot_general(
        qbd_ref[:, :], k_ref[:, :], (((1,), (1,)), ((), ())),
        preferred_element_type=jnp.float32,
    ) * (D ** -0.5)
    s = jnp.where(cnt > 0.0, s, NEG)
    m = jnp.max(s, axis=1, keepdims=True)
    p = jnp.exp(s - m) * cnt
    l = jnp.sum(p, axis=1, keepdims=True)

    o_full = lax.dot_general(
        p.astype(jnp.bfloat16), v_ref[:, :], (((1,), (0,)), ((), ())),
        preferred_element_type=jnp.float32,
    )
    colh = lax.div(lax.broadcasted_iota(jnp.int32, (R, HD), 1), D)
    rowh = lax.rem(lax.broadcasted_iota(jnp.int32, (R, HD), 0), H)
    o_m = jnp.where(colh == rowh, o_full, 0.0)
    o = o_m[:, 0:D]
    for h in range(1, H):
        o = o + o_m[:, h * D:(h + 1) * D]

    mine_ref[:, 0:D] = o
    mine_ref[:, D:D + 1] = m
    mine_ref[:, D + 1:D + 2] = l

    descs = []
    for k in (1, 2, 3):
        r = 3 - k
        rdma = pltpu.make_async_remote_copy(
            src_ref=mine_ref,
            dst_ref=comm_ref.at[r],
            send_sem=send_sems.at[k - 1],
            recv_sem=recv_sems.at[r],
            device_id=((my + k) % N_DEV,),
            device_id_type=pl.DeviceIdType.MESH,
        )
        rdma.start()
        descs.append(rdma)
    for d_ in descs:
        d_.wait_send()
    for d_ in descs:
        d_.wait_recv()

    mine = mine_ref[:, :]
    m_g = mine[:, D:D + 1]
    for r in range(3):
        m_g = jnp.maximum(m_g, comm_ref[r, :, D:D + 1])
    acc = mine * jnp.exp(mine[:, D:D + 1] - m_g)
    for r in range(3):
        part = comm_ref[r, :, :]
        acc = acc + part * jnp.exp(part[:, D:D + 1] - m_g)
    out = acc[:, 0:D] / acc[:, D + 1:D + 2]
    out_ref[:, 0, :, :] = out.reshape(B, H, D)


def kernel(Q, K, V, bt, lens):
    my = lax.axis_index("i")
    off = my * P_LOCAL

    ids = off + jnp.arange(P_LOCAL, dtype=jnp.int32)
    valid = jnp.arange(NB, dtype=jnp.int32)[None, :] < lens[:, None]
    eq = bt[:, :, None] == ids[None, None, :]
    counts = jnp.sum(
        jnp.where(eq & valid[:, :, None], 1.0, 0.0), axis=1
    )
    cnt_keys = jnp.repeat(counts, BS, axis=1).astype(jnp.bfloat16)

    qbd = jnp.einsum(
        "bhd,gh->bghd", Q[:, 0], jnp.eye(H, dtype=Q.dtype)
    ).reshape(R, HD).astype(jnp.bfloat16)
    k2 = K.reshape(NKEYS, HD).astype(jnp.bfloat16)
    v2 = V.reshape(NKEYS, HD).astype(jnp.bfloat16)

    return pl.pallas_call(
        _body,
        out_shape=jax.ShapeDtypeStruct((B, 1, H, D), jnp.float32),
        in_specs=[pl.BlockSpec(memory_space=pltpu.VMEM)] * 4,
        out_specs=pl.BlockSpec(memory_space=pltpu.VMEM),
        scratch_shapes=[
            pltpu.VMEM((R, 128), jnp.float32),
            pltpu.VMEM((3, R, 128), jnp.float32),
            pltpu.SemaphoreType.DMA((3,)),
            pltpu.SemaphoreType.DMA((3,)),
        ],
        compiler_params=pltpu.CompilerParams(collective_id=0),
    )(qbd, k2, v2, cnt_keys)


# device time: 33718 ns/iter; 1.6607x vs baseline; 1.6607x over previous
import jax
import jax.numpy as jnp
from jax import lax
from jax.experimental import pallas as pl
from jax.experimental.pallas import tpu as pltpu

N_DEV = 4
B = 16
H = 16
D = 64
BS = 16
NB = 128
P_LOCAL = 128
NKEYS = P_LOCAL * BS
NEG = -1e30


def _body(q_ref, k_ref, v_ref, cnt_ref, out_ref,
          mine_ref, comm_ref, send_sems, recv_sems):
    my = lax.axis_index("i")

    bsem = pltpu.get_barrier_semaphore()
    for k in (1, 2, 3):
        pl.semaphore_signal(
            bsem, inc=1,
            device_id=((my + k) % N_DEV,),
            device_id_type=pl.DeviceIdType.MESH,
        )

    cnt = cnt_ref[:, :]
    scale = D ** -0.5

    def compute_head(h):
        c, j = divmod(h, 8)
        q_h = q_ref[h, :, :]
        k_h = k_ref[h, :, :]
        s = lax.dot_general(
            q_h, k_h, (((1,), (1,)), ((), ())),
            preferred_element_type=jnp.float32,
        ) * scale
        s = jnp.where(cnt > 0.0, s, NEG)
        m = jnp.max(s, axis=1, keepdims=True)
        p = jnp.exp(s - m) * cnt
        l = jnp.sum(p, axis=1, keepdims=True)
        v_h = v_ref[h, :, :]
        o = lax.dot_general(
            p.astype(jnp.bfloat16), v_h, (((1,), (0,)), ((), ())),
            preferred_element_type=jnp.float32,
        )
        mine_ref[c, :, j, 0:D] = o
        mine_ref[c, :, j, D:D + 1] = m
        mine_ref[c, :, j, D + 1:D + 2] = l

    def send_chunk(c):
        descs = []
        for k in (1, 2, 3):
            r = 3 - k
            rdma = pltpu.make_async_remote_copy(
                src_ref=mine_ref.at[c],
                dst_ref=comm_ref.at[r, c],
                send_sem=send_sems.at[k - 1, c],
                recv_sem=recv_sems.at[r, c],
                device_id=((my + k) % N_DEV,),
                device_id_type=pl.DeviceIdType.MESH,
            )
            rdma.start()
            descs.append(rdma)
        return descs

    def combine_chunk(c):
        mine = mine_ref[c, :, :, :]
        m_g = mine[:, :, D:D + 1]
        for r in range(3):
            m_g = jnp.maximum(m_g, comm_ref[r, c, :, :, D:D + 1])
        acc = mine * jnp.exp(mine[:, :, D:D + 1] - m_g)
        for r in range(3):
            part = comm_ref[r, c, :, :, :]
            acc = acc + part * jnp.exp(part[:, :, D:D + 1] - m_g)
        out_ref[:, 0, c * 8:(c + 1) * 8, :] = (
            acc[:, :, 0:D] / acc[:, :, D + 1:D + 2])

    for h in range(8):
        compute_head(h)
    pl.semaphore_wait(bsem, N_DEV - 1)
    descs0 = send_chunk(0)
    for h in range(8, 16):
        compute_head(h)
    descs1 = send_chunk(1)

    for d in descs0:
        d.wait_recv()
    combine_chunk(0)
    for d in descs1:
        d.wait_recv()
    combine_chunk(1)
    for d in descs0 + descs1:
        d.wait_send()


def kernel(Q, K, V, bt, lens):
    my = lax.axis_index("i")
    off = my * P_LOCAL

    ids = off + jnp.arange(P_LOCAL, dtype=jnp.int32)
    valid = jnp.arange(NB, dtype=jnp.int32)[None, :] < lens[:, None]
    eq = bt[:, :, None] == ids[None, None, :]
    counts = jnp.sum(
        jnp.where(eq & valid[:, :, None], 1.0, 0.0), axis=1
    ).astype(jnp.float32)
    cnt_keys = jnp.repeat(counts, BS, axis=1)
    qs = Q[:, 0].swapaxes(0, 1).astype(jnp.bfloat16)
    kt = K.reshape(NKEYS, H, D).swapaxes(0, 1).astype(jnp.bfloat16)
    vt = V.reshape(NKEYS, H, D).swapaxes(0, 1).astype(jnp.bfloat16)

    return pl.pallas_call(
        _body,
        out_shape=jax.ShapeDtypeStruct((B, 1, H, D), jnp.float32),
        in_specs=[pl.BlockSpec(memory_space=pltpu.VMEM)] * 4,
        out_specs=pl.BlockSpec(memory_space=pltpu.VMEM),
        scratch_shapes=[
            pltpu.VMEM((2, B, 8, 128), jnp.float32),
            pltpu.VMEM((3, 2, B, 8, 128), jnp.float32),
            pltpu.SemaphoreType.DMA((3, 2)),
            pltpu.SemaphoreType.DMA((3, 2)),
        ],
        compiler_params=pltpu.CompilerParams(collective_id=0),
    )(qs, kt, vt, cnt_keys)
